# baseline (speedup 1.0000x reference)
import functools

import jax
import jax.numpy as jnp
from jax.experimental import pallas as pl
from jax.experimental.pallas import tpu as pltpu

_BM = 400


def _s1_kernel(x_ref, w1_ref, out_ref):
    out_ref[...] = jnp.dot(x_ref[...], w1_ref[...],
                           preferred_element_type=jnp.float32)


def _h_kernel(adj_ref, s1_ref, b1_ref, out_ref):
    acc = jnp.dot(adj_ref[...], s1_ref[...],
                  preferred_element_type=jnp.float32)
    out_ref[...] = jnp.maximum(acc + b1_ref[...], 0.0)


def _o_kernel(adj_ref, h_ref, w2_ref, b2_ref, out_ref):
    t = jnp.dot(adj_ref[...], h_ref[...], preferred_element_type=jnp.float32)
    o = jnp.dot(t, w2_ref[...], preferred_element_type=jnp.float32)
    o = o + b2_ref[...]
    m = jnp.max(o, axis=1, keepdims=True)
    shifted = o - m
    lse = jnp.log(jnp.sum(jnp.exp(shifted), axis=1, keepdims=True))
    out_ref[...] = shifted - lse


@jax.jit
def kernel(x, adj, W1, b1, W2, b2):
    n, nfeat = x.shape
    nhid = W1.shape[1]
    nclass = W2.shape[1]
    b1r = b1.reshape(1, nhid)
    b2r = b2.reshape(1, nclass)
    nblocks = n // _BM

    s1 = pl.pallas_call(
        _s1_kernel,
        out_shape=jax.ShapeDtypeStruct((n, nhid), jnp.float32),
    )(x, W1)

    h = pl.pallas_call(
        _h_kernel,
        grid=(nblocks,),
        in_specs=[
            pl.BlockSpec((_BM, n), lambda i: (i, 0)),
            pl.BlockSpec((n, nhid), lambda i: (0, 0)),
            pl.BlockSpec((1, nhid), lambda i: (0, 0)),
        ],
        out_specs=pl.BlockSpec((_BM, nhid), lambda i: (i, 0)),
        out_shape=jax.ShapeDtypeStruct((n, nhid), jnp.float32),
        compiler_params=pltpu.CompilerParams(
            dimension_semantics=("parallel",)),
    )(adj, s1, b1r)

    o = pl.pallas_call(
        _o_kernel,
        grid=(nblocks,),
        in_specs=[
            pl.BlockSpec((_BM, n), lambda i: (i, 0)),
            pl.BlockSpec((n, nhid), lambda i: (0, 0)),
            pl.BlockSpec((nhid, nclass), lambda i: (0, 0)),
            pl.BlockSpec((1, nclass), lambda i: (0, 0)),
        ],
        out_specs=pl.BlockSpec((_BM, nclass), lambda i: (i, 0)),
        out_shape=jax.ShapeDtypeStruct((n, nclass), jnp.float32),
        compiler_params=pltpu.CompilerParams(
            dimension_semantics=("parallel",)),
    )(adj, h, W2, b2r)

    return o


# speedup vs baseline: 1.0622x; 1.0622x over previous
import jax
import jax.numpy as jnp
from jax.experimental import pallas as pl
from jax.experimental.pallas import tpu as pltpu

_BM = 400


def _fused_kernel(x_ref, adj_ref, w1_ref, b1_ref, w2_ref, b2_ref,
                  out_ref, s1_ref, h2_ref):
    i = pl.program_id(0)
    nb = pl.num_programs(0) // 2

    @pl.when(i == 0)
    def _():
        s1_ref[...] = jnp.dot(x_ref[...], w1_ref[...],
                              preferred_element_type=jnp.float32)

    @pl.when(i < nb)
    def _():
        acc = jnp.dot(adj_ref[...], s1_ref[...],
                      preferred_element_type=jnp.float32)
        hb = jnp.maximum(acc + b1_ref[...], 0.0)
        h2_ref[pl.ds(i * _BM, _BM), :] = jnp.dot(
            hb, w2_ref[...], preferred_element_type=jnp.float32)

    @pl.when(i >= nb)
    def _():
        o = jnp.dot(adj_ref[...], h2_ref[...],
                    preferred_element_type=jnp.float32)
        o = o + b2_ref[...]
        m = jnp.max(o, axis=1, keepdims=True)
        shifted = o - m
        lse = jnp.log(jnp.sum(jnp.exp(shifted), axis=1, keepdims=True))
        out_ref[...] = shifted - lse


@jax.jit
def kernel(x, adj, W1, b1, W2, b2):
    n, nfeat = x.shape
    nhid = W1.shape[1]
    nclass = W2.shape[1]
    b1r = b1.reshape(1, nhid)
    b2r = b2.reshape(1, nclass)
    nb = n // _BM

    return pl.pallas_call(
        _fused_kernel,
        grid=(2 * nb,),
        in_specs=[
            pl.BlockSpec((n, nfeat), lambda i: (0, 0)),
            pl.BlockSpec((_BM, n), lambda i: (i % nb, 0)),
            pl.BlockSpec((nfeat, nhid), lambda i: (0, 0)),
            pl.BlockSpec((1, nhid), lambda i: (0, 0)),
            pl.BlockSpec((nhid, nclass), lambda i: (0, 0)),
            pl.BlockSpec((1, nclass), lambda i: (0, 0)),
        ],
        out_specs=pl.BlockSpec(
            (_BM, nclass), lambda i: (jnp.maximum(i - nb, 0), 0)),
        out_shape=jax.ShapeDtypeStruct((n, nclass), jnp.float32),
        scratch_shapes=[
            pltpu.VMEM((n, nhid), jnp.float32),
            pltpu.VMEM((n, nclass), jnp.float32),
        ],
        compiler_params=pltpu.CompilerParams(
            dimension_semantics=("arbitrary",)),
    )(x, adj, W1, b1r, W2, b2r)
